# Initial kernel scaffold; baseline (speedup 1.0000x reference)
import jax
import jax.numpy as jnp
from jax.experimental import pallas as pl


def kernel(input_tensor, random_R, kernel_total, ln_gamma, ln_beta, W1, b1, W2, b2):
    raise NotImplementedError("write your pallas kernel here")


# speedup vs baseline: 3.7694x; 3.7694x over previous
import jax
import jax.numpy as jnp
from jax import lax
from jax.experimental import pallas as pl

_B, _T, _C = 4, 2048, 768
_K = 8
_FF = 4 * _C
_NKEY = 10
_LK = 16
_BLK = 256


def _prestage_body(x_ref, rext_ref, w_ref, hid_ref):
    T, LK = _T, _LK
    x = x_ref[0]
    h = jnp.dot(x, rext_ref[...], preferred_element_type=jnp.float32)
    lane = lax.broadcasted_iota(jnp.int32, (T, LK), 1)
    hm = jnp.where(lane < _NKEY, h, jnp.float32(-1e30))
    rowmax = jnp.max(hm, axis=1, keepdims=True)
    idxf = jnp.min(jnp.where(hm >= rowmax, lane, LK), axis=1, keepdims=True)
    onehot = (lane == idxf).astype(jnp.float32)

    r128 = lax.broadcasted_iota(jnp.int32, (128, 128), 0)
    c128 = lax.broadcasted_iota(jnp.int32, (128, 128), 1)
    tril = (c128 <= r128).astype(jnp.bfloat16)
    pieces = []
    run = jnp.zeros((1, LK), jnp.float32)
    for p in range(T // 128):
        blk = onehot[p * 128:(p + 1) * 128, :].astype(jnp.bfloat16)
        cin = jnp.dot(tril, blk, preferred_element_type=jnp.float32)
        pieces.append(cin + run)
        run = run + cin[127:128, :]
    cum = jnp.concatenate(pieces, axis=0)
    counts = run
    cum_excl = cum - onehot
    below = jnp.sum(jnp.where(lane < idxf, jnp.broadcast_to(counts, (T, LK)), 0.0),
                    axis=1, keepdims=True)
    within = jnp.sum(onehot * cum_excl, axis=1, keepdims=True)
    rank = below + within

    v = x[:, 0:1]
    cols = []
    for rb in range(T // 128):
        tgt = (lax.broadcasted_iota(jnp.int32, (T, 128), 1) + rb * 128).astype(jnp.float32)
        hit = rank == tgt
        cols.append(jnp.sum(jnp.where(hit, jnp.broadcast_to(v, (T, 128)), 0.0),
                            axis=0, keepdims=True))
    g = jnp.concatenate(cols, axis=1)

    wv = w_ref[0:1, :]
    hid = g * (wv[:, 0:1] + wv[:, _K:_K + 1])
    for j in range(1, _K):
        sh = jnp.concatenate([jnp.zeros((1, j), jnp.float32), g[:, :T - j]], axis=1)
        hid = hid + sh * wv[:, j:j + 1]
    hid_ref[0] = hid


def _prestage(x, rext, wpad):
    return pl.pallas_call(
        _prestage_body,
        grid=(_B,),
        in_specs=[
            pl.BlockSpec((1, _T, _C), lambda b: (b, 0, 0)),
            pl.BlockSpec((_C, _LK), lambda b: (0, 0)),
            pl.BlockSpec((1, _LK), lambda b: (0, 0)),
        ],
        out_specs=pl.BlockSpec((1, 1, _T), lambda b: (b, 0, 0)),
        out_shape=jax.ShapeDtypeStruct((_B, 1, _T), jnp.float32),
    )(x, rext, wpad)


def _ffn_body(x_ref, hid_ref, gam_ref, bet_ref, w1_ref, b1_ref, w2_ref, b2_ref, o_ref):
    x = x_ref[...]
    out = x + hid_ref[...]
    mu = jnp.mean(out, axis=1, keepdims=True)
    d = out - mu
    var = jnp.mean(d * d, axis=1, keepdims=True)
    y = d * lax.rsqrt(var + 1e-5) * gam_ref[...] + bet_ref[...]
    h1 = lax.dot_general(y, w1_ref[...], (((1,), (1,)), ((), ())),
                         preferred_element_type=jnp.float32) + b1_ref[...]
    h1 = 0.5 * h1 * (1.0 + lax.erf(h1 * jnp.float32(0.7071067811865476)))
    h2 = lax.dot_general(h1, w2_ref[...], (((1,), (1,)), ((), ())),
                         preferred_element_type=jnp.float32) + b2_ref[...]
    o_ref[...] = h2 + out


def _ffn(x2, hid2, gam2, bet2, w1, b12, w2, b22):
    n = _B * _T
    return pl.pallas_call(
        _ffn_body,
        grid=(n // _BLK,),
        in_specs=[
            pl.BlockSpec((_BLK, _C), lambda i: (i, 0)),
            pl.BlockSpec((_BLK, 1), lambda i: (i, 0)),
            pl.BlockSpec((1, _C), lambda i: (0, 0)),
            pl.BlockSpec((1, _C), lambda i: (0, 0)),
            pl.BlockSpec((_FF, _C), lambda i: (0, 0)),
            pl.BlockSpec((1, _FF), lambda i: (0, 0)),
            pl.BlockSpec((_C, _FF), lambda i: (0, 0)),
            pl.BlockSpec((1, _C), lambda i: (0, 0)),
        ],
        out_specs=pl.BlockSpec((_BLK, _C), lambda i: (i, 0)),
        out_shape=jax.ShapeDtypeStruct((n, _C), jnp.float32),
    )(x2, hid2, gam2, bet2, w1, b12, w2, b22)


def kernel(input_tensor, random_R, kernel_total, ln_gamma, ln_beta, W1, b1, W2, b2):
    x = input_tensor
    rext = jnp.concatenate(
        [random_R, -random_R, jnp.zeros((_C, _LK - 2 * 5), jnp.float32)], axis=1)
    wpad = jnp.pad(kernel_total.reshape(1, _K + 1), ((0, 0), (0, _LK - (_K + 1))))
    hid = _prestage(x, rext, wpad)
    out = _ffn(x.reshape(_B * _T, _C), hid.reshape(_B * _T, 1),
               ln_gamma.reshape(1, _C), ln_beta.reshape(1, _C),
               W1, b1.reshape(1, _FF), W2, b2.reshape(1, _C))
    return out.reshape(_B, _T, _C)
